# baseline (speedup 1.0000x reference)
import functools

import jax
import jax.numpy as jnp
from jax import lax
from jax.experimental import pallas as pl
from jax.experimental.pallas import tpu as pltpu
from jax.experimental.pallas import tpu_sc as plsc

VOCAB = 100000
EMBED = 64
MAX_NORM = 1.0
B = 1024
L = 20
N_ROWS = B * L

_NC, _NS = 2, 16
NW = _NC * _NS
ROWS_PER_W = N_ROWS // NW
CHUNK = 128
N_CHUNKS = ROWS_PER_W // CHUNK

VT = 2048
GRID = -(-VOCAB // VT)


@functools.lru_cache(maxsize=1)
def _make_sc_gather():
    @functools.partial(
        pl.kernel,
        mesh=plsc.VectorSubcoreMesh(core_axis_name="c", subcore_axis_name="s"),
        out_type=jax.ShapeDtypeStruct((N_ROWS, EMBED), jnp.float32),
        scratch_types=[
            pltpu.VMEM((N_CHUNKS, CHUNK), jnp.int32),
            pltpu.VMEM((ROWS_PER_W, EMBED), jnp.float32),
            pltpu.SemaphoreType.DMA,
        ],
        compiler_params=pltpu.CompilerParams(use_tc_tiling_on_sc=False),
    )
    def _sc_gather(idx_hbm, table_hbm, emb_hbm, idx_v, rows_v, sem):
        wid = lax.axis_index("s") * _NC + lax.axis_index("c")
        pltpu.sync_copy(idx_hbm.at[wid], idx_v)
        copies = []
        for k in range(N_CHUNKS):
            copies.append(
                pltpu.async_copy(
                    table_hbm.at[idx_v.at[k]],
                    rows_v.at[pl.ds(k * CHUNK, CHUNK)],
                    sem,
                )
            )
        for c in copies:
            c.wait()
        pltpu.sync_copy(rows_v, emb_hbm.at[pl.ds(wid * ROWS_PER_W, ROWS_PER_W)])

    return _sc_gather


def _tc_body(emb_ref, w_ref, b_ref, out_ref, x_ref):
    @pl.when(pl.program_id(0) == 0)
    def _():
        e = emb_ref[...]
        ss = jnp.sum(e * e, axis=1, keepdims=True)
        norm = jnp.sqrt(ss)
        scale = jnp.minimum(1.0, MAX_NORM / jnp.maximum(norm, 1e-7))
        es = (e * scale).reshape(B, L, EMBED)
        x_ref[...] = jnp.sum(es, axis=1) * (1.0 / L)

    out_ref[...] = lax.dot_general(
        x_ref[...], w_ref[...], (((1,), (1,)), ((), ())),
        preferred_element_type=jnp.float32,
    ) + b_ref[...]


_tc_call = pl.pallas_call(
    _tc_body,
    grid=(GRID,),
    in_specs=[
        pl.BlockSpec((N_ROWS, EMBED), lambda i: (0, 0)),
        pl.BlockSpec((VT, EMBED), lambda i: (i, 0)),
        pl.BlockSpec((1, VT), lambda i: (0, i)),
    ],
    out_specs=pl.BlockSpec((B, VT), lambda i: (0, i)),
    out_shape=jax.ShapeDtypeStruct((B, VOCAB), jnp.float32),
    scratch_shapes=[pltpu.VMEM((B, EMBED), jnp.float32)],
)


def kernel(inputs_, table, W, b):
    idx = inputs_.astype(jnp.int32).reshape(NW, N_CHUNKS, CHUNK)
    emb = _make_sc_gather()(idx, table)
    return _tc_call(emb, W, b.reshape(1, VOCAB))


# speedup vs baseline: 1.0030x; 1.0030x over previous
import functools

import jax
import jax.numpy as jnp
from jax import lax
from jax.experimental import pallas as pl
from jax.experimental.pallas import tpu as pltpu
from jax.experimental.pallas import tpu_sc as plsc

VOCAB = 100000
EMBED = 64
MAX_NORM = 1.0
B = 1024
L = 20
N_ROWS = B * L

_NC, _NS = 2, 16
NW = _NC * _NS
ROWS_PER_W = N_ROWS // NW
CHUNK = 128
N_CHUNKS = ROWS_PER_W // CHUNK

VT = 2048
GRID = -(-VOCAB // VT)


@functools.lru_cache(maxsize=1)
def _make_sc_gather():
    @functools.partial(
        pl.kernel,
        mesh=plsc.VectorSubcoreMesh(core_axis_name="c", subcore_axis_name="s"),
        out_type=jax.ShapeDtypeStruct((N_ROWS, EMBED), jnp.float32),
        scratch_types=[
            pltpu.VMEM((N_CHUNKS, CHUNK), jnp.int32),
            pltpu.VMEM((ROWS_PER_W, EMBED), jnp.float32),
            pltpu.SemaphoreType.DMA,
        ],
        compiler_params=pltpu.CompilerParams(use_tc_tiling_on_sc=False),
    )
    def _sc_gather(idx_hbm, table_hbm, emb_hbm, idx_v, rows_v, sem):
        wid = lax.axis_index("s") * _NC + lax.axis_index("c")
        pltpu.sync_copy(idx_hbm.at[wid], idx_v)
        copies = []
        for k in range(N_CHUNKS):
            copies.append(
                pltpu.async_copy(
                    table_hbm.at[idx_v.at[k]],
                    rows_v.at[pl.ds(k * CHUNK, CHUNK)],
                    sem,
                )
            )
        for c in copies:
            c.wait()
        pltpu.sync_copy(rows_v, emb_hbm.at[pl.ds(wid * ROWS_PER_W, ROWS_PER_W)])

    return _sc_gather


def _renorm_body(emb_ref, x_ref):
    e = emb_ref[...]
    ss = jnp.sum(e * e, axis=1, keepdims=True)
    norm = jnp.sqrt(ss)
    scale = jnp.minimum(1.0, MAX_NORM / jnp.maximum(norm, 1e-7))
    es = (e * scale).reshape(B, L, EMBED)
    x_ref[...] = jnp.sum(es, axis=1) * (1.0 / L)


_renorm_call = pl.pallas_call(
    _renorm_body,
    out_shape=jax.ShapeDtypeStruct((B, EMBED), jnp.float32),
)


def _mm_body(x_ref, w_ref, b_ref, out_ref):
    out_ref[...] = lax.dot_general(
        x_ref[...], w_ref[...], (((1,), (1,)), ((), ())),
        preferred_element_type=jnp.float32,
    ) + b_ref[...]


_mm_call = pl.pallas_call(
    _mm_body,
    grid=(GRID,),
    in_specs=[
        pl.BlockSpec((B, EMBED), lambda i: (0, 0)),
        pl.BlockSpec((VT, EMBED), lambda i: (i, 0)),
        pl.BlockSpec((1, VT), lambda i: (0, i)),
    ],
    out_specs=pl.BlockSpec((B, VT), lambda i: (0, i)),
    out_shape=jax.ShapeDtypeStruct((B, VOCAB), jnp.float32),
)


def kernel(inputs_, table, W, b):
    idx = inputs_.astype(jnp.int32).reshape(NW, N_CHUNKS, CHUNK)
    emb = _make_sc_gather()(idx, table)
    x = _renorm_call(emb)
    return _mm_call(x, W, b.reshape(1, VOCAB))
